# baseline (speedup 1.0000x reference)
import functools

import jax
import jax.numpy as jnp
import numpy as np
from jax.experimental import pallas as pl
from jax.experimental.pallas import tpu as pltpu
from jax.experimental.pallas import tpu_sc as plsc

F32 = jnp.float32
_BN_DIV = np.float32(np.sqrt(np.float32(1.0 + 1e-5)))
_INTERPRET = False


def _fps_body(xt_ref, out_ref, *, npoint):
    x = xt_ref[...]
    B, _, N = x.shape
    iota_n = jax.lax.broadcasted_iota(jnp.int32, (B, N), 1)
    iota_s = jax.lax.broadcasted_iota(jnp.int32, (B, 3, npoint), 2)

    def step(i, carry):
        dist_min, far, acc = carry
        onehot = (iota_n == far).astype(F32)[:, None, :]
        c = jnp.sum(x * onehot, axis=2, keepdims=True)
        acc = jnp.where(iota_s == i, c, acc)
        d = x - c
        dist = d[:, 0, :] * d[:, 0, :] + d[:, 1, :] * d[:, 1, :] + d[:, 2, :] * d[:, 2, :]
        dist_min = jnp.minimum(dist_min, dist)
        m = jnp.max(dist_min, axis=1, keepdims=True)
        far = jnp.min(jnp.where(dist_min == m, iota_n, N), axis=1, keepdims=True)
        return dist_min, far, acc

    init = (
        jnp.full((B, N), 1e10, F32),
        jnp.zeros((B, 1), jnp.int32),
        jnp.zeros((B, 3, npoint), F32),
    )
    _, _, acc = jax.lax.fori_loop(0, npoint, step, init)
    out_ref[...] = acc


def _fps(xt, npoint):
    B, _, N = xt.shape
    return pl.pallas_call(
        functools.partial(_fps_body, npoint=npoint),
        out_shape=jax.ShapeDtypeStruct((B, 3, npoint), F32),
        interpret=_INTERPRET,
    )(xt)


def _bq_body(src_ref, xt_ref, idx_ref, *, radius2, nsample):
    src = src_ref[0]
    xt = xt_ref[0]
    St = src.shape[0]
    N = xt.shape[1]
    mm = jnp.dot(src, xt)
    s2 = jnp.sum(src * src, axis=1, keepdims=True)
    d2 = jnp.sum(xt * xt, axis=0, keepdims=True)
    sqr = (s2 + d2) - 2.0 * mm
    mask = jnp.logical_not(sqr > radius2)
    nc = N // 128
    r128 = jax.lax.broadcasted_iota(jnp.int32, (128, 128), 0)
    c128 = jax.lax.broadcasted_iota(jnp.int32, (128, 128), 1)
    u128 = (r128 <= c128).astype(F32)
    rc = jax.lax.broadcasted_iota(jnp.int32, (nc, nc), 0)
    cc = jax.lax.broadcasted_iota(jnp.int32, (nc, nc), 1)
    uc = (rc < cc).astype(F32)
    m2 = mask.astype(F32).reshape(St * nc, 128)
    y = jnp.dot(m2, u128)
    tot = y[:, 127:128].reshape(St, nc)
    offs = jnp.dot(tot, uc)
    rank = (y.reshape(St, nc, 128) + offs[:, :, None]).reshape(St, N).astype(jnp.int32)
    count = rank[:, N - 1 : N]
    cols = []
    j0 = None
    for k in range(nsample):
        jk = jnp.sum(
            jnp.where(rank <= k, 1.0, 0.0), axis=1, keepdims=True
        ).astype(jnp.int32)
        if k == 0:
            j0 = jk
            cols.append(jk)
        else:
            cols.append(jnp.where(count > k, jk, j0))
    idx_ref[0] = jnp.concatenate(cols, axis=1)


def _bq(new_xyz, xt, radius2, nsample, s_tile):
    B, S, _ = new_xyz.shape
    N = xt.shape[2]
    return pl.pallas_call(
        functools.partial(_bq_body, radius2=radius2, nsample=nsample),
        grid=(B, S // s_tile),
        in_specs=[
            pl.BlockSpec((1, s_tile, 3), lambda b, s: (b, s, 0)),
            pl.BlockSpec((1, 3, N), lambda b, s: (b, 0, 0)),
        ],
        out_specs=pl.BlockSpec((1, s_tile, nsample), lambda b, s: (b, s, 0)),
        out_shape=jax.ShapeDtypeStruct((B, S, nsample), jnp.int32),
        interpret=_INTERPRET,
    )(new_xyz, xt)


def _sa_body(g_ref, c_ref, *refs, ns, nlayers):
    o_ref = refs[-1]
    w_refs = refs[:-1]
    g = g_ref[...]
    c = c_ref[...]
    Rt, Cpad = g.shape
    Ct = c.shape[0]
    cpad = jnp.concatenate([c, jnp.zeros((Ct, Cpad - 3), F32)], axis=1)
    crep = jnp.broadcast_to(cpad[:, None, :], (Ct, ns, Cpad)).reshape(Rt, Cpad)
    h = g - crep
    for li in range(nlayers):
        wt, b, gm, be = (w_refs[4 * li + j][...] for j in range(4))
        h = jnp.dot(h, wt) + b
        h = jnp.maximum(h / _BN_DIV * gm + be, 0.0)
    Cout = h.shape[1]
    o_ref[...] = jnp.max(h.reshape(Ct, ns, Cout), axis=1)


def _sa_mlp(g, centers, layers, ns, c_tile):
    R, Cpad = g.shape
    M = R // ns
    nlayers = len(layers)
    cout = layers[-1][0].shape[1]
    w_args = []
    w_specs = []
    for wt, b, gm, be in layers:
        for a in (wt, b, gm, be):
            w_args.append(a)
            w_specs.append(pl.BlockSpec(a.shape, lambda i: (0,) * a.ndim))
    return pl.pallas_call(
        functools.partial(_sa_body, ns=ns, nlayers=nlayers),
        grid=(M // c_tile,),
        in_specs=[
            pl.BlockSpec((c_tile * ns, Cpad), lambda i: (i, 0)),
            pl.BlockSpec((c_tile, 3), lambda i: (i, 0)),
        ]
        + w_specs,
        out_specs=pl.BlockSpec((c_tile, cout), lambda i: (i, 0)),
        out_shape=jax.ShapeDtypeStruct((M, cout), F32),
        interpret=_INTERPRET,
    )(g, centers, *w_args)


def _fp_sel_body(x1_ref, x2t_ref, idx_ref, w_ref, *, n2):
    x1 = x1_ref[0]
    x2t = x2t_ref[0]
    Nt = x1.shape[0]
    mm = jnp.dot(x1, x2t)
    s2 = jnp.sum(x1 * x1, axis=1, keepdims=True)
    d2 = jnp.sum(x2t * x2t, axis=0, keepdims=True)
    sqr = (s2 + d2) - 2.0 * mm
    iota = jax.lax.broadcasted_iota(jnp.int32, (Nt, n2), 1)
    d = sqr
    ms, iks = [], []
    for _ in range(3):
        m = jnp.min(d, axis=1, keepdims=True)
        ik = jnp.min(jnp.where(d == m, iota, n2), axis=1, keepdims=True)
        d = jnp.where(iota == ik, 3.4e38, d)
        ms.append(m)
        iks.append(ik)
    recs = [1.0 / (m + 1e-8) for m in ms]
    rsum = (recs[0] + recs[1]) + recs[2]
    base = pl.program_id(0) * n2
    idx_ref[0] = jnp.concatenate(iks, axis=1) + base
    w_ref[0] = jnp.concatenate([r / rsum for r in recs], axis=1)


def _fp_sel(xyz1, x2t, n_tile):
    B, N1, _ = xyz1.shape
    N2 = x2t.shape[2]
    return pl.pallas_call(
        functools.partial(_fp_sel_body, n2=N2),
        grid=(B, N1 // n_tile),
        in_specs=[
            pl.BlockSpec((1, n_tile, 3), lambda b, n: (b, n, 0)),
            pl.BlockSpec((1, 3, N2), lambda b, n: (b, 0, 0)),
        ],
        out_specs=[
            pl.BlockSpec((1, n_tile, 3), lambda b, n: (b, n, 0)),
            pl.BlockSpec((1, n_tile, 3), lambda b, n: (b, n, 0)),
        ],
        out_shape=[
            jax.ShapeDtypeStruct((B, N1, 3), jnp.int32),
            jax.ShapeDtypeStruct((B, N1, 3), F32),
        ],
        interpret=_INTERPRET,
    )(xyz1, x2t)


def _fp_mlp_body(g_ref, w_ref, p1_ref, *refs, nlayers):
    o_ref = refs[-1]
    w_refs = refs[:-1]
    g = g_ref[...]
    wts = w_ref[0]
    p1 = p1_ref[0]
    Nt = p1.shape[0]
    C2 = g.shape[1]
    g3 = g.reshape(Nt, 3, C2)
    interp = jnp.sum(g3 * wts[:, :, None], axis=1)
    h = jnp.concatenate([p1, interp], axis=1)
    for li in range(nlayers):
        wt, b, gm, be = (w_refs[4 * li + j][...] for j in range(4))
        h = jnp.dot(h, wt) + b
        h = jnp.maximum(h / _BN_DIV * gm + be, 0.0)
    o_ref[0] = h


def _fp_mlp(g, w, p1, layers, n_tile):
    B, N1, _ = w.shape
    C1 = p1.shape[2]
    C2 = g.shape[1]
    nlayers = len(layers)
    cout = layers[-1][0].shape[1]
    w_args = []
    w_specs = []
    nblk = N1 // n_tile
    for wt, b, gm, be in layers:
        for a in (wt, b, gm, be):
            w_args.append(a)
            w_specs.append(pl.BlockSpec(a.shape, lambda b_, n_: (0,) * a.ndim))
    return pl.pallas_call(
        functools.partial(_fp_mlp_body, nlayers=nlayers),
        grid=(B, nblk),
        in_specs=[
            pl.BlockSpec((n_tile * 3, C2), lambda b, n: (b * nblk + n, 0)),
            pl.BlockSpec((1, n_tile, 3), lambda b, n: (b, n, 0)),
            pl.BlockSpec((1, n_tile, C1), lambda b, n: (b, n, 0)),
        ]
        + w_specs,
        out_specs=pl.BlockSpec((1, n_tile, cout), lambda b, n: (b, n, 0)),
        out_shape=jax.ShapeDtypeStruct((B, N1, cout), F32),
        interpret=_INTERPRET,
    )(g, w, p1, *w_args)


def _fp(xyz1, x2t, p1, p2, layers, n_tile):
    B, N1, _ = xyz1.shape
    N2 = x2t.shape[2]
    C2 = p2.shape[2]
    idxg, w = _fp_sel(xyz1, x2t, n_tile)
    rows = _gather_rows(p2.reshape(B * N2, C2), idxg.reshape(-1))
    return _fp_mlp(rows, w, p1, layers, n_tile)


def _head_body(l0_ref, xyz_ref, w1, b1, g1, be1, w2, b2, o_ref):
    l0 = l0_ref[0]
    g = jnp.mean(l0, axis=0, keepdims=True)
    comb = jnp.concatenate([l0, jnp.broadcast_to(g, l0.shape)], axis=1)
    x = jnp.dot(comb, w1[...]) + b1[...]
    x = jnp.maximum(x / _BN_DIV * g1[...] + be1[...], 0.0)
    lo = jnp.dot(x, w2[...]) + b2[...]
    o_ref[0] = jnp.concatenate([xyz_ref[0], lo], axis=1)


def _head(l0_out, xyz0, hw):
    B, N, C = l0_out.shape
    w1, b1, g1, be1, w2, b2 = hw
    w_specs = [pl.BlockSpec(a.shape, lambda b: (0,) * a.ndim) for a in hw]
    return pl.pallas_call(
        _head_body,
        grid=(B,),
        in_specs=[
            pl.BlockSpec((1, N, C), lambda b: (b, 0, 0)),
            pl.BlockSpec((1, N, 3), lambda b: (b, 0, 0)),
        ]
        + w_specs,
        out_specs=pl.BlockSpec((1, N, 4), lambda b: (b, 0, 0)),
        out_shape=jax.ShapeDtypeStruct((B, N, 4), F32),
        interpret=_INTERPRET,
    )(l0_out, xyz0, *hw)


_NWORKERS = 32


def _gather_rows(table, idx_flat):
    R = idx_flat.shape[0]
    width = table.shape[1]
    per = R // _NWORKERS
    chunks = 1
    while (per // chunks) * width * 4 > 256 * 1024 or (per // chunks) > 8192:
        chunks *= 2
    cper = per // chunks
    mesh = plsc.VectorSubcoreMesh(core_axis_name="c", subcore_axis_name="s")

    @functools.partial(
        pl.kernel,
        mesh=mesh,
        compiler_params=pltpu.CompilerParams(use_tc_tiling_on_sc=False),
        out_type=jax.ShapeDtypeStruct((R, width), F32),
        scratch_types=[
            pltpu.VMEM((cper,), jnp.int32),
            pltpu.VMEM((cper, width), F32),
            pltpu.SemaphoreType.DMA,
        ],
    )
    def k(table_hbm, idx_hbm, out_hbm, idx_v, rows_v, sem):
        wid = jax.lax.axis_index("s") * 2 + jax.lax.axis_index("c")
        base = wid * per
        for ci in range(chunks):
            off = base + ci * cper
            pltpu.sync_copy(idx_hbm.at[pl.ds(off, cper)], idx_v)
            pltpu.async_copy(table_hbm.at[idx_v], rows_v, sem).wait()
            pltpu.sync_copy(rows_v, out_hbm.at[pl.ds(off, cper)])

    return k(table, idx_flat)


def _prep_layers(layers, cpad=None):
    out = []
    cin = None
    for wt, b, gm, be in layers:
        w = jnp.transpose(wt)
        if cpad is not None and cin is None and w.shape[0] < cpad:
            w = jnp.concatenate([w, jnp.zeros((cpad - w.shape[0], w.shape[1]), F32)], axis=0)
        cin = w.shape[0]
        out.append((w, b.reshape(1, -1), gm.reshape(1, -1), be.reshape(1, -1)))
    return out


def kernel(coords, features, params):
    B, N0, _ = coords.shape
    xyz0 = coords[..., :3]
    x0t = jnp.transpose(xyz0, (0, 2, 1))

    S1, NS = 1024, 32
    nx1t = _fps(x0t, S1)
    new_xyz1 = jnp.transpose(nx1t, (0, 2, 1))
    idx1 = _bq(new_xyz1, x0t, 0.25, NS, 256)
    src1 = jnp.concatenate(
        [xyz0, features, jnp.zeros((B, N0, 9), F32)], axis=-1
    ).reshape(B * N0, 16)
    gidx1 = (idx1 + (jnp.arange(B, dtype=jnp.int32) * N0)[:, None, None]).reshape(-1)
    g1 = _gather_rows(src1, gidx1)
    sa1 = _prep_layers(params["sa1"], cpad=16)
    l1 = _sa_mlp(g1, new_xyz1.reshape(B * S1, 3), sa1, NS, 64)
    l1_points = l1.reshape(B, S1, 64)

    S2 = 256
    nx2t = _fps(nx1t, S2)
    new_xyz2 = jnp.transpose(nx2t, (0, 2, 1))
    idx2 = _bq(new_xyz2, nx1t, 1.0, NS, 256)
    src2 = jnp.concatenate(
        [new_xyz1, l1_points, jnp.zeros((B, S1, 13), F32)], axis=-1
    ).reshape(B * S1, 80)
    gidx2 = (idx2 + (jnp.arange(B, dtype=jnp.int32) * S1)[:, None, None]).reshape(-1)
    g2 = _gather_rows(src2, gidx2)
    sa2 = _prep_layers(params["sa2"], cpad=80)
    l2 = _sa_mlp(g2, new_xyz2.reshape(B * S2, 3), sa2, NS, 64)
    l2_points = l2.reshape(B, S2, 128)

    fp2 = _prep_layers(params["fp2"])
    l1_new = _fp(new_xyz1, nx2t, l1_points, l2_points, fp2, 512)

    fp1 = _prep_layers(params["fp1"])
    l0_new = _fp(xyz0, nx1t, features, l1_new, fp1, 512)

    w1, b1, g1, be1 = params["head_conv1"]
    w2, b2 = params["head_conv2"]
    hw = (
        jnp.transpose(w1),
        b1.reshape(1, -1),
        g1.reshape(1, -1),
        be1.reshape(1, -1),
        jnp.transpose(w2),
        b2.reshape(1, -1),
    )
    return _head(l0_new, xyz0, hw)


# speedup vs baseline: 1.0576x; 1.0576x over previous
import functools

import jax
import jax.numpy as jnp
import numpy as np
from jax.experimental import pallas as pl
from jax.experimental.pallas import tpu as pltpu
from jax.experimental.pallas import tpu_sc as plsc

F32 = jnp.float32
_BN_DIV = np.float32(np.sqrt(np.float32(1.0 + 1e-5)))


def _fps_body(xt_ref, out_ref, *, npoint):
    x = xt_ref[...]
    B, _, N = x.shape
    iota_n = jax.lax.broadcasted_iota(jnp.int32, (B, N), 1)
    iota_s = jax.lax.broadcasted_iota(jnp.int32, (B, 3, npoint), 2)

    def step(i, carry):
        dist_min, far, acc = carry
        onehot = (iota_n == far).astype(F32)[:, None, :]
        c = jnp.sum(x * onehot, axis=2, keepdims=True)
        acc = jnp.where(iota_s == i, c, acc)
        d = x - c
        dist = d[:, 0, :] * d[:, 0, :] + d[:, 1, :] * d[:, 1, :] + d[:, 2, :] * d[:, 2, :]
        dist_min = jnp.minimum(dist_min, dist)
        m = jnp.max(dist_min, axis=1, keepdims=True)
        far = jnp.min(jnp.where(dist_min == m, iota_n, N), axis=1, keepdims=True)
        return dist_min, far, acc

    init = (
        jnp.full((B, N), 1e10, F32),
        jnp.zeros((B, 1), jnp.int32),
        jnp.zeros((B, 3, npoint), F32),
    )
    _, _, acc = jax.lax.fori_loop(0, npoint, step, init)
    out_ref[...] = acc


def _fps(xt, npoint):
    B, _, N = xt.shape
    return pl.pallas_call(
        functools.partial(_fps_body, npoint=npoint),
        out_shape=jax.ShapeDtypeStruct((B, 3, npoint), F32),
    )(xt)


def _bq_body(src_ref, xt_ref, idx_ref, *, radius2, nsample):
    src = src_ref[0]
    xt = xt_ref[0]
    St = src.shape[0]
    N = xt.shape[1]
    mm = jnp.dot(src, xt)
    s2 = jnp.sum(src * src, axis=1, keepdims=True)
    d2 = jnp.sum(xt * xt, axis=0, keepdims=True)
    sqr = (s2 + d2) - 2.0 * mm
    mask = jnp.logical_not(sqr > radius2)
    nc = N // 128
    r128 = jax.lax.broadcasted_iota(jnp.int32, (128, 128), 0)
    c128 = jax.lax.broadcasted_iota(jnp.int32, (128, 128), 1)
    u128 = (r128 <= c128).astype(F32)
    rc = jax.lax.broadcasted_iota(jnp.int32, (nc, nc), 0)
    cc = jax.lax.broadcasted_iota(jnp.int32, (nc, nc), 1)
    uc = (rc < cc).astype(F32)
    m2 = mask.astype(F32).reshape(St * nc, 128)
    y = jnp.dot(m2, u128)
    tot = y[:, 127:128].reshape(St, nc)
    offs = jnp.dot(tot, uc)
    rank = (y.reshape(St, nc, 128) + offs[:, :, None]).reshape(St, N).astype(jnp.int32)
    count = rank[:, N - 1 : N]
    cols = []
    j0 = None
    for k in range(nsample):
        jk = jnp.sum(
            jnp.where(rank <= k, 1.0, 0.0), axis=1, keepdims=True
        ).astype(jnp.int32)
        if k == 0:
            j0 = jk
            cols.append(jk)
        else:
            cols.append(jnp.where(count > k, jk, j0))
    idx_ref[0] = jnp.concatenate(cols, axis=1)


def _bq(new_xyz, xt, radius2, nsample, s_tile):
    B, S, _ = new_xyz.shape
    N = xt.shape[2]
    return pl.pallas_call(
        functools.partial(_bq_body, radius2=radius2, nsample=nsample),
        grid=(B, S // s_tile),
        in_specs=[
            pl.BlockSpec((1, s_tile, 3), lambda b, s: (b, s, 0)),
            pl.BlockSpec((1, 3, N), lambda b, s: (b, 0, 0)),
        ],
        out_specs=pl.BlockSpec((1, s_tile, nsample), lambda b, s: (b, s, 0)),
        out_shape=jax.ShapeDtypeStruct((B, S, nsample), jnp.int32),
    )(new_xyz, xt)


def _sa_body(g_ref, c_ref, *refs, ns, nlayers):
    o_ref = refs[-1]
    w_refs = refs[:-1]
    g = g_ref[...]
    c = c_ref[...]
    Rt, Cpad = g.shape
    Ct = c.shape[0]
    cpad = jnp.concatenate([c, jnp.zeros((Ct, Cpad - 3), F32)], axis=1)
    crep = jnp.broadcast_to(cpad[:, None, :], (Ct, ns, Cpad)).reshape(Rt, Cpad)
    h = g - crep
    for li in range(nlayers):
        wt, b, gm, be = (w_refs[4 * li + j][...] for j in range(4))
        h = jnp.dot(h, wt) + b
        h = jnp.maximum(h / _BN_DIV * gm + be, 0.0)
    Cout = h.shape[1]
    o_ref[...] = jnp.max(h.reshape(Ct, ns, Cout), axis=1)


def _sa_mlp(g, centers, layers, ns, c_tile):
    R, Cpad = g.shape
    M = R // ns
    nlayers = len(layers)
    cout = layers[-1][0].shape[1]
    w_args = []
    w_specs = []
    for wt, b, gm, be in layers:
        for a in (wt, b, gm, be):
            w_args.append(a)
            w_specs.append(pl.BlockSpec(a.shape, lambda i: (0,) * a.ndim))
    return pl.pallas_call(
        functools.partial(_sa_body, ns=ns, nlayers=nlayers),
        grid=(M // c_tile,),
        in_specs=[
            pl.BlockSpec((c_tile * ns, Cpad), lambda i: (i, 0)),
            pl.BlockSpec((c_tile, 3), lambda i: (i, 0)),
        ]
        + w_specs,
        out_specs=pl.BlockSpec((c_tile, cout), lambda i: (i, 0)),
        out_shape=jax.ShapeDtypeStruct((M, cout), F32),
    )(g, centers, *w_args)


def _fp_body(x1_ref, x2t_ref, p1_ref, p2_ref, *refs, nlayers):
    o_ref = refs[-1]
    w_refs = refs[:-1]
    x1 = x1_ref[0]
    x2t = x2t_ref[0]
    p1 = p1_ref[0]
    p2 = p2_ref[0]
    Nt = x1.shape[0]
    N2 = x2t.shape[1]
    mm = jnp.dot(x1, x2t)
    s2 = jnp.sum(x1 * x1, axis=1, keepdims=True)
    d2 = jnp.sum(x2t * x2t, axis=0, keepdims=True)
    sqr = (s2 + d2) - 2.0 * mm
    iota = jax.lax.broadcasted_iota(jnp.int32, (Nt, N2), 1)
    d = sqr
    ms, iks = [], []
    for _ in range(3):
        m = jnp.min(d, axis=1, keepdims=True)
        ik = jnp.min(jnp.where(d == m, iota, N2), axis=1, keepdims=True)
        d = jnp.where(iota == ik, 3.4e38, d)
        ms.append(m)
        iks.append(ik)
    recs = [1.0 / (m + 1e-8) for m in ms]
    rsum = (recs[0] + recs[1]) + recs[2]
    wm = jnp.zeros((Nt, N2), F32)
    for m, ik, rec in zip(ms, iks, recs):
        wm = wm + jnp.where(iota == ik, rec / rsum, 0.0)
    interp = jnp.dot(wm, p2)
    h = jnp.concatenate([p1, interp], axis=1)
    for li in range(nlayers):
        wt, b, gm, be = (w_refs[4 * li + j][...] for j in range(4))
        h = jnp.dot(h, wt) + b
        h = jnp.maximum(h / _BN_DIV * gm + be, 0.0)
    o_ref[0] = h


def _fp(xyz1, x2t, p1, p2, layers, n_tile):
    B, N1, _ = xyz1.shape
    N2 = x2t.shape[2]
    C1 = p1.shape[2]
    C2 = p2.shape[2]
    nlayers = len(layers)
    cout = layers[-1][0].shape[1]
    w_args = []
    w_specs = []
    for wt, b, gm, be in layers:
        for a in (wt, b, gm, be):
            w_args.append(a)
            w_specs.append(pl.BlockSpec(a.shape, lambda b_, n_: (0,) * a.ndim))
    return pl.pallas_call(
        functools.partial(_fp_body, nlayers=nlayers),
        grid=(B, N1 // n_tile),
        in_specs=[
            pl.BlockSpec((1, n_tile, 3), lambda b, n: (b, n, 0)),
            pl.BlockSpec((1, 3, N2), lambda b, n: (b, 0, 0)),
            pl.BlockSpec((1, n_tile, C1), lambda b, n: (b, n, 0)),
            pl.BlockSpec((1, N2, C2), lambda b, n: (b, 0, 0)),
        ]
        + w_specs,
        out_specs=pl.BlockSpec((1, n_tile, cout), lambda b, n: (b, n, 0)),
        out_shape=jax.ShapeDtypeStruct((B, N1, cout), F32),
    )(xyz1, x2t, p1, p2, *w_args)


def _head_body(l0_ref, xyz_ref, w1, b1, g1, be1, w2, b2, o_ref):
    l0 = l0_ref[0]
    g = jnp.mean(l0, axis=0, keepdims=True)
    comb = jnp.concatenate([l0, jnp.broadcast_to(g, l0.shape)], axis=1)
    x = jnp.dot(comb, w1[...]) + b1[...]
    x = jnp.maximum(x / _BN_DIV * g1[...] + be1[...], 0.0)
    lo = jnp.dot(x, w2[...]) + b2[...]
    o_ref[0] = jnp.concatenate([xyz_ref[0], lo], axis=1)


def _head(l0_out, xyz0, hw):
    B, N, C = l0_out.shape
    w1, b1, g1, be1, w2, b2 = hw
    w_specs = [pl.BlockSpec(a.shape, lambda b: (0,) * a.ndim) for a in hw]
    return pl.pallas_call(
        _head_body,
        grid=(B,),
        in_specs=[
            pl.BlockSpec((1, N, C), lambda b: (b, 0, 0)),
            pl.BlockSpec((1, N, 3), lambda b: (b, 0, 0)),
        ]
        + w_specs,
        out_specs=pl.BlockSpec((1, N, 4), lambda b: (b, 0, 0)),
        out_shape=jax.ShapeDtypeStruct((B, N, 4), F32),
    )(l0_out, xyz0, *hw)


_NWORKERS = 32


def _gather_rows(table, idx_flat):
    R = idx_flat.shape[0]
    width = table.shape[1]
    per = R // _NWORKERS
    chunks = 1
    while (per // chunks) * width * 4 > 256 * 1024 or (per // chunks) > 8192:
        chunks *= 2
    cper = per // chunks
    mesh = plsc.VectorSubcoreMesh(core_axis_name="c", subcore_axis_name="s")

    @functools.partial(
        pl.kernel,
        mesh=mesh,
        compiler_params=pltpu.CompilerParams(use_tc_tiling_on_sc=False),
        out_type=jax.ShapeDtypeStruct((R, width), F32),
        scratch_types=[
            pltpu.VMEM((cper,), jnp.int32),
            pltpu.VMEM((cper, width), F32),
            pltpu.SemaphoreType.DMA,
        ],
    )
    def k(table_hbm, idx_hbm, out_hbm, idx_v, rows_v, sem):
        wid = jax.lax.axis_index("s") * 2 + jax.lax.axis_index("c")
        base = wid * per
        for ci in range(chunks):
            off = base + ci * cper
            pltpu.sync_copy(idx_hbm.at[pl.ds(off, cper)], idx_v)
            pltpu.async_copy(table_hbm.at[idx_v], rows_v, sem).wait()
            pltpu.sync_copy(rows_v, out_hbm.at[pl.ds(off, cper)])

    return k(table, idx_flat)


def _prep_layers(layers, cpad=None):
    out = []
    cin = None
    for wt, b, gm, be in layers:
        w = jnp.transpose(wt)
        if cpad is not None and cin is None and w.shape[0] < cpad:
            w = jnp.concatenate([w, jnp.zeros((cpad - w.shape[0], w.shape[1]), F32)], axis=0)
        cin = w.shape[0]
        out.append((w, b.reshape(1, -1), gm.reshape(1, -1), be.reshape(1, -1)))
    return out


def kernel(coords, features, params):
    B, N0, _ = coords.shape
    xyz0 = coords[..., :3]
    x0t = jnp.transpose(xyz0, (0, 2, 1))

    S1, NS = 1024, 32
    nx1t = _fps(x0t, S1)
    new_xyz1 = jnp.transpose(nx1t, (0, 2, 1))
    idx1 = _bq(new_xyz1, x0t, 0.25, NS, 256)
    src1 = jnp.concatenate(
        [xyz0, features, jnp.zeros((B, N0, 9), F32)], axis=-1
    ).reshape(B * N0, 16)
    gidx1 = (idx1 + (jnp.arange(B, dtype=jnp.int32) * N0)[:, None, None]).reshape(-1)
    g1 = _gather_rows(src1, gidx1)
    sa1 = _prep_layers(params["sa1"], cpad=16)
    l1 = _sa_mlp(g1, new_xyz1.reshape(B * S1, 3), sa1, NS, 64)
    l1_points = l1.reshape(B, S1, 64)

    S2 = 256
    nx2t = _fps(nx1t, S2)
    new_xyz2 = jnp.transpose(nx2t, (0, 2, 1))
    idx2 = _bq(new_xyz2, nx1t, 1.0, NS, 256)
    src2 = jnp.concatenate(
        [new_xyz1, l1_points, jnp.zeros((B, S1, 13), F32)], axis=-1
    ).reshape(B * S1, 80)
    gidx2 = (idx2 + (jnp.arange(B, dtype=jnp.int32) * S1)[:, None, None]).reshape(-1)
    g2 = _gather_rows(src2, gidx2)
    sa2 = _prep_layers(params["sa2"], cpad=80)
    l2 = _sa_mlp(g2, new_xyz2.reshape(B * S2, 3), sa2, NS, 64)
    l2_points = l2.reshape(B, S2, 128)

    fp2 = _prep_layers(params["fp2"])
    l1_new = _fp(new_xyz1, nx2t, l1_points, l2_points, fp2, 512)

    fp1 = _prep_layers(params["fp1"])
    l0_new = _fp(xyz0, nx1t, features, l1_new, fp1, 512)

    w1, b1, g1, be1 = params["head_conv1"]
    w2, b2 = params["head_conv2"]
    hw = (
        jnp.transpose(w1),
        b1.reshape(1, -1),
        g1.reshape(1, -1),
        be1.reshape(1, -1),
        jnp.transpose(w2),
        b2.reshape(1, -1),
    )
    return _head(l0_new, xyz0, hw)
